# baseline (speedup 1.0000x reference)
import jax
import jax.numpy as jnp
from jax import lax
from jax.experimental import pallas as pl
from jax.experimental.pallas import tpu as pltpu
from jax.experimental.pallas import tpu_sc as plsc

E = 8
D_MODEL = 1024
HIDDEN = 2048
T = 2048
CAP = 512
NSLOT = CAP - 1
TRASH = NSLOT

NC = 2
NS = 16
EPC = E // NC
SCR = EPC * CAP
TPT = T // NS
CHUNK = 64

_sc_mesh = plsc.VectorSubcoreMesh(core_axis_name="c", subcore_axis_name="s")


def _routing_kernel(x_ref, w_ref, b_ref, idx_ref, a0_ref, a1_ref, bb_ref,
                    s0_ref, s1_ref):
    x = x_ref[...]
    w = w_ref[...]
    scores = lax.dot_general(
        w, x, (((0,), (1,)), ((), ())),
        preferred_element_type=jnp.float32,
    ) + b_ref[...].reshape(E, 1)

    eidx = lax.broadcasted_iota(jnp.int32, (E, T), 0)
    v0 = jnp.max(scores, axis=0, keepdims=True)
    i0 = jnp.min(jnp.where(scores == v0, eidx, E), axis=0, keepdims=True)
    masked = jnp.where(eidx == i0, -jnp.inf, scores)
    v1 = jnp.max(masked, axis=0, keepdims=True)
    i1 = jnp.min(jnp.where(masked == v1, eidx, E), axis=0, keepdims=True)
    ed = jnp.exp(v1 - v0)
    denom = 1.0 + ed
    s0 = 1.0 / denom
    s1 = ed / denom

    oh0 = (eidx == i0).astype(jnp.float32)
    oh1 = (eidx == i1).astype(jnp.float32)
    i0 = i0.astype(jnp.float32)
    i1 = i1.astype(jnp.float32)
    c = jnp.concatenate([oh0, oh1], axis=0)
    k = 1
    while k < T:
        shifted = jnp.concatenate(
            [jnp.zeros((2 * E, k), jnp.float32), c[:, : T - k]], axis=1)
        c = c + shifted
        k *= 2
    pos0 = c[:E, :]
    pos1 = pos0 + c[E:, :]
    p0 = jnp.sum(oh0 * pos0, axis=0, keepdims=True)
    p1 = jnp.sum(oh1 * pos1, axis=0, keepdims=True)
    m0 = p0 < float(CAP)
    m1 = p1 < float(CAP)

    r0 = jnp.where(m0, i0 * float(CAP) + p0 - 1.0, float(TRASH))
    r1 = jnp.where(m1, i1 * float(CAP) + p1 - 1.0, float(TRASH))
    a0 = jnp.where(m0, s0, 0.0)
    a1 = jnp.where(m1, s1, 0.0)
    bb = jnp.where(m0, 0.0, s0) + jnp.where(m1, 0.0, s1)

    idx_ref[...] = jnp.concatenate([r0, r1], axis=0).astype(jnp.int32)
    for ref, val in ((a0_ref, a0), (a1_ref, a1), (bb_ref, bb)):
        ref[...] = jnp.broadcast_to(lax.transpose(val, (1, 0)), (T, 16))

    tw = (float(T)
          - lax.broadcasted_iota(jnp.int32, (1, T), 1).astype(jnp.float32))
    for a in range(E * CAP // 128):
        rid = (lax.broadcasted_iota(jnp.int32, (128, 1), 0).astype(jnp.float32)
               + float(a * 128))
        s0 = float(T) - jnp.sum(jnp.where(rid == r0, tw, 0.0),
                                axis=1, keepdims=True)
        s1 = float(T) - jnp.sum(jnp.where(rid == r1, tw, 0.0),
                                axis=1, keepdims=True)
        sl = pl.ds(a * 128, 128)
        s0_ref[sl, :] = jnp.clip(s0, 0.0, float(T)).astype(jnp.int32)
        s1_ref[sl, :] = jnp.clip(s1, 0.0, float(T)).astype(jnp.int32)


def _routing(x2d, router_w, router_b):
    return pl.pallas_call(
        _routing_kernel,
        out_shape=(
            jax.ShapeDtypeStruct((2, T), jnp.int32),
            jax.ShapeDtypeStruct((T, 16), jnp.float32),
            jax.ShapeDtypeStruct((T, 16), jnp.float32),
            jax.ShapeDtypeStruct((T, 16), jnp.float32),
            jax.ShapeDtypeStruct((E * CAP, 1), jnp.int32),
            jax.ShapeDtypeStruct((E * CAP, 1), jnp.int32),
        ),
    )(x2d, router_w, router_b)


RPT = E * CAP // (NC * NS)
DCH = 16


def _dispatch_body(xe_hbm, s0_hbm, s1_hbm, o0_hbm, o1_hbm,
                   i0v, i1v, g0a, g0b, g1a, g1b, sem0, sem1):
    c = lax.axis_index("c")
    s = lax.axis_index("s")
    base = (s * NC + c) * RPT
    nch = RPT // DCH
    g0 = (g0a, g0b)
    g1 = (g1a, g1b)
    sems = (sem0, sem1)

    pltpu.sync_copy(s0_hbm.at[pl.ds(base, RPT)], i0v)
    pltpu.sync_copy(s1_hbm.at[pl.ds(base, RPT)], i1v)

    def fire(chunk):
        p = chunk % 2
        sl = pl.ds(chunk * DCH, DCH)
        c0 = pltpu.async_copy(xe_hbm.at[i0v.at[sl]], g0[p], sems[p])
        c1 = pltpu.async_copy(xe_hbm.at[i1v.at[sl]], g1[p], sems[p])
        return c0, c1

    pend = fire(0)
    for chunk in range(nch):
        p = chunk % 2
        cur = pend
        if chunk + 1 < nch:
            pend = fire(chunk + 1)
        cur[0].wait()
        cur[1].wait()
        r0 = base + chunk * DCH
        pltpu.sync_copy(g0[p], o0_hbm.at[pl.ds(r0, DCH)])
        pltpu.sync_copy(g1[p], o1_hbm.at[pl.ds(r0, DCH)])


def _dispatch(x_ext, src0, src1):
    return pl.kernel(
        _dispatch_body,
        out_type=(
            jax.ShapeDtypeStruct((E * CAP, D_MODEL), jnp.float32),
            jax.ShapeDtypeStruct((E * CAP, D_MODEL), jnp.float32),
        ),
        mesh=_sc_mesh,
        scratch_types=[
            pltpu.VMEM((RPT,), jnp.int32),
            pltpu.VMEM((RPT,), jnp.int32),
            pltpu.VMEM((DCH, D_MODEL), jnp.float32),
            pltpu.VMEM((DCH, D_MODEL), jnp.float32),
            pltpu.VMEM((DCH, D_MODEL), jnp.float32),
            pltpu.VMEM((DCH, D_MODEL), jnp.float32),
            pltpu.SemaphoreType.DMA,
            pltpu.SemaphoreType.DMA,
        ],
    )(x_ext, src0, src1)


HB = 1024
NHB = HIDDEN // HB


def _ff_kernel(g0_ref, g1_ref, w1_ref, w2_ref, w3_ref, out_ref):
    hb = pl.program_id(1)
    g = (g0_ref[...] + g1_ref[...]).astype(jnp.bfloat16)
    w1b = w1_ref[0].astype(jnp.bfloat16)
    w2b = w2_ref[0].astype(jnp.bfloat16)
    w3b = w3_ref[0].astype(jnp.bfloat16)
    h = jnp.dot(g, w2b, preferred_element_type=jnp.float32) * jnp.dot(
        g, w1b, preferred_element_type=jnp.float32)
    h = jax.nn.gelu(h).astype(jnp.bfloat16)
    part = jnp.dot(h, w3b, preferred_element_type=jnp.float32)

    @pl.when(hb == 0)
    def _init():
        out_ref[...] = part

    @pl.when(hb != 0)
    def _acc():
        out_ref[...] += part


def _expert_ff(g0, g1, w1, w2, w3):
    return pl.pallas_call(
        _ff_kernel,
        grid=(E, NHB),
        in_specs=[
            pl.BlockSpec((CAP, D_MODEL), lambda e, h: (e, 0)),
            pl.BlockSpec((CAP, D_MODEL), lambda e, h: (e, 0)),
            pl.BlockSpec((1, D_MODEL, HB), lambda e, h: (e, 0, h)),
            pl.BlockSpec((1, D_MODEL, HB), lambda e, h: (e, 0, h)),
            pl.BlockSpec((1, HB, D_MODEL), lambda e, h: (e, h, 0)),
        ],
        out_specs=pl.BlockSpec((CAP, D_MODEL), lambda e, h: (e, 0)),
        out_shape=jax.ShapeDtypeStruct((E * CAP, D_MODEL), jnp.float32),
    )(g0, g1, w1, w2, w3)


def _combine_body(eo_hbm, x_hbm, c0_hbm, c1_hbm, a0_hbm, a1_hbm, bb_hbm,
                  out_hbm, xb, g0b, g1b, i0v, i1v, a0v, a1v, bbv, sem):
    c = lax.axis_index("c")
    s = lax.axis_index("s")
    wid = s * NC + c
    base = wid * (T // (NC * NS))
    for chunk in range(2):
        t0 = base + chunk * 32
        pltpu.sync_copy(x_hbm.at[pl.ds(t0, 32)], xb)
        pltpu.sync_copy(c0_hbm.at[pl.ds(t0, 32)], i0v)
        pltpu.sync_copy(c1_hbm.at[pl.ds(t0, 32)], i1v)
        pltpu.sync_copy(a0_hbm.at[pl.ds(t0, 32)], a0v)
        pltpu.sync_copy(a1_hbm.at[pl.ds(t0, 32)], a1v)
        pltpu.sync_copy(bb_hbm.at[pl.ds(t0, 32)], bbv)
        pltpu.async_copy(eo_hbm.at[i0v], g0b, sem).wait()
        pltpu.async_copy(eo_hbm.at[i1v], g1b, sem).wait()

        def row(i, carry):
            a0 = a0v[i, pl.ds(0, 16)]
            a1 = a1v[i, pl.ds(0, 16)]
            bb = bbv[i, pl.ds(0, 16)]
            for j in range(D_MODEL // 16):
                sl = pl.ds(j * 16, 16)
                xb[i, sl] = (a0 * g0b[i, sl] + a1 * g1b[i, sl]
                             + bb * xb[i, sl])
            return carry

        lax.fori_loop(0, 32, row, 0)
        pltpu.sync_copy(xb, out_hbm.at[pl.ds(t0, 32)])


def _combine(eo, x2d, comb0, comb1, a0, a1, bb):
    return pl.kernel(
        _combine_body,
        out_type=jax.ShapeDtypeStruct((T, D_MODEL), jnp.float32),
        mesh=_sc_mesh,
        scratch_types=[
            pltpu.VMEM((32, D_MODEL), jnp.float32),
            pltpu.VMEM((32, D_MODEL), jnp.float32),
            pltpu.VMEM((32, D_MODEL), jnp.float32),
            pltpu.VMEM((32,), jnp.int32),
            pltpu.VMEM((32,), jnp.int32),
            pltpu.VMEM((32, 16), jnp.float32),
            pltpu.VMEM((32, 16), jnp.float32),
            pltpu.VMEM((32, 16), jnp.float32),
            pltpu.SemaphoreType.DMA,
        ],
    )(eo, x2d, comb0, comb1, a0, a1, bb)


def kernel(x, router_w, router_b, w1, w2, w3):
    x2d = x.reshape(T, D_MODEL)
    idx, a0, a1, bb, src0, src1 = _routing(x2d, router_w, router_b)
    x_ext = jnp.concatenate([x2d, jnp.zeros((1, D_MODEL), jnp.float32)])
    g0, g1 = _dispatch(x_ext, src0.reshape(-1), src1.reshape(-1))
    eo = _expert_ff(g0, g1, w1, w2, w3)
    out = _combine(eo, x2d, idx[0], idx[1], a0, a1, bb)
    return out.reshape(1, T, D_MODEL)


# speedup vs baseline: 1.8468x; 1.8468x over previous
import jax
import jax.numpy as jnp
from jax import lax
from jax.experimental import pallas as pl
from jax.experimental.pallas import tpu as pltpu
from jax.experimental.pallas import tpu_sc as plsc

E = 8
D_MODEL = 1024
HIDDEN = 2048
T = 2048
CAP = 512
NSLOT = CAP - 1
TRASH = NSLOT

NC = 2
NS = 16
EPC = E // NC
SCR = EPC * CAP
TPT = T // NS
CHUNK = 64

_sc_mesh = plsc.VectorSubcoreMesh(core_axis_name="c", subcore_axis_name="s")


def _routing_kernel(x_ref, w_ref, b_ref, idx_ref, a0_ref, a1_ref, bb_ref,
                    s0_ref, s1_ref):
    x = x_ref[...]
    w = w_ref[...]
    scores = lax.dot_general(
        w, x, (((0,), (1,)), ((), ())),
        preferred_element_type=jnp.float32,
    ) + b_ref[...].reshape(E, 1)

    eidx = lax.broadcasted_iota(jnp.int32, (E, T), 0)
    v0 = jnp.max(scores, axis=0, keepdims=True)
    i0 = jnp.min(jnp.where(scores == v0, eidx, E), axis=0, keepdims=True)
    masked = jnp.where(eidx == i0, -jnp.inf, scores)
    v1 = jnp.max(masked, axis=0, keepdims=True)
    i1 = jnp.min(jnp.where(masked == v1, eidx, E), axis=0, keepdims=True)
    ed = jnp.exp(v1 - v0)
    denom = 1.0 + ed
    s0 = 1.0 / denom
    s1 = ed / denom

    oh0 = (eidx == i0).astype(jnp.float32)
    oh1 = (eidx == i1).astype(jnp.float32)
    i0 = i0.astype(jnp.float32)
    i1 = i1.astype(jnp.float32)
    c = jnp.concatenate([oh0, oh1], axis=0)
    k = 1
    while k < T:
        shifted = jnp.concatenate(
            [jnp.zeros((2 * E, k), jnp.float32), c[:, : T - k]], axis=1)
        c = c + shifted
        k *= 2
    pos0 = c[:E, :]
    pos1 = pos0 + c[E:, :]
    p0 = jnp.sum(oh0 * pos0, axis=0, keepdims=True)
    p1 = jnp.sum(oh1 * pos1, axis=0, keepdims=True)
    m0 = p0 < float(CAP)
    m1 = p1 < float(CAP)

    r0 = jnp.where(m0, i0 * float(CAP) + p0 - 1.0, float(TRASH))
    r1 = jnp.where(m1, i1 * float(CAP) + p1 - 1.0, float(TRASH))
    a0 = jnp.where(m0, s0, 0.0)
    a1 = jnp.where(m1, s1, 0.0)
    bb = jnp.where(m0, 0.0, s0) + jnp.where(m1, 0.0, s1)

    idx_ref[...] = jnp.concatenate([r0, r1], axis=0).astype(jnp.int32)
    for ref, val in ((a0_ref, a0), (a1_ref, a1), (bb_ref, bb)):
        ref[...] = jnp.broadcast_to(lax.transpose(val, (1, 0)), (T, 16))

    tw = (float(T)
          - lax.broadcasted_iota(jnp.int32, (1, T), 1).astype(jnp.float32))
    for a in range(E * CAP // 128):
        rid = (lax.broadcasted_iota(jnp.int32, (128, 1), 0).astype(jnp.float32)
               + float(a * 128))
        s0 = float(T) - jnp.sum(jnp.where(rid == r0, tw, 0.0),
                                axis=1, keepdims=True)
        s1 = float(T) - jnp.sum(jnp.where(rid == r1, tw, 0.0),
                                axis=1, keepdims=True)
        s0 = jnp.clip(s0, 0.0, float(T))
        s1 = jnp.clip(s1, 0.0, float(T))
        local = rid - float(a * 128)
        s0 = jnp.where(s0 == float(T), s0 + local, s0)
        s1 = jnp.where(s1 == float(T), s1 + local, s1)
        sl = pl.ds(a * 128, 128)
        s0_ref[sl, :] = s0.astype(jnp.int32)
        s1_ref[sl, :] = s1.astype(jnp.int32)


def _routing(x2d, router_w, router_b):
    return pl.pallas_call(
        _routing_kernel,
        out_shape=(
            jax.ShapeDtypeStruct((2, T), jnp.int32),
            jax.ShapeDtypeStruct((T, 16), jnp.float32),
            jax.ShapeDtypeStruct((T, 16), jnp.float32),
            jax.ShapeDtypeStruct((T, 16), jnp.float32),
            jax.ShapeDtypeStruct((E * CAP, 1), jnp.int32),
            jax.ShapeDtypeStruct((E * CAP, 1), jnp.int32),
        ),
    )(x2d, router_w, router_b)


RPT = E * CAP // (NC * NS)
DCH = 16


def _dispatch_body(xe_hbm, s0_hbm, s1_hbm, o0_hbm, o1_hbm,
                   i0v, i1v, g0a, g0b, g1a, g1b, sem0, sem1):
    c = lax.axis_index("c")
    s = lax.axis_index("s")
    base = (s * NC + c) * RPT
    nch = RPT // DCH
    g0 = (g0a, g0b)
    g1 = (g1a, g1b)
    sems = (sem0, sem1)

    pltpu.sync_copy(s0_hbm.at[pl.ds(base, RPT)], i0v)
    pltpu.sync_copy(s1_hbm.at[pl.ds(base, RPT)], i1v)

    def fire(chunk):
        p = chunk % 2
        sl = pl.ds(chunk * DCH, DCH)
        c0 = pltpu.async_copy(xe_hbm.at[i0v.at[sl]], g0[p], sems[p])
        c1 = pltpu.async_copy(xe_hbm.at[i1v.at[sl]], g1[p], sems[p])
        return c0, c1

    pend = fire(0)
    for chunk in range(nch):
        p = chunk % 2
        cur = pend
        if chunk + 1 < nch:
            pend = fire(chunk + 1)
        cur[0].wait()
        cur[1].wait()
        r0 = base + chunk * DCH
        pltpu.sync_copy(g0[p], o0_hbm.at[pl.ds(r0, DCH)])
        pltpu.sync_copy(g1[p], o1_hbm.at[pl.ds(r0, DCH)])


def _dispatch(x_ext, src0, src1):
    return pl.kernel(
        _dispatch_body,
        out_type=(
            jax.ShapeDtypeStruct((E * CAP, D_MODEL), jnp.float32),
            jax.ShapeDtypeStruct((E * CAP, D_MODEL), jnp.float32),
        ),
        mesh=_sc_mesh,
        scratch_types=[
            pltpu.VMEM((RPT,), jnp.int32),
            pltpu.VMEM((RPT,), jnp.int32),
            pltpu.VMEM((DCH, D_MODEL), jnp.float32),
            pltpu.VMEM((DCH, D_MODEL), jnp.float32),
            pltpu.VMEM((DCH, D_MODEL), jnp.float32),
            pltpu.VMEM((DCH, D_MODEL), jnp.float32),
            pltpu.SemaphoreType.DMA,
            pltpu.SemaphoreType.DMA,
        ],
    )(x_ext, src0, src1)


HB = 1024
NHB = HIDDEN // HB


def _ff_kernel(g0_ref, g1_ref, w1_ref, w2_ref, w3_ref, out_ref):
    hb = pl.program_id(1)
    g = (g0_ref[...] + g1_ref[...]).astype(jnp.bfloat16)
    w1b = w1_ref[0].astype(jnp.bfloat16)
    w2b = w2_ref[0].astype(jnp.bfloat16)
    w3b = w3_ref[0].astype(jnp.bfloat16)
    h = jnp.dot(g, w2b, preferred_element_type=jnp.float32) * jnp.dot(
        g, w1b, preferred_element_type=jnp.float32)
    h = jax.nn.gelu(h).astype(jnp.bfloat16)
    part = jnp.dot(h, w3b, preferred_element_type=jnp.float32)

    @pl.when(hb == 0)
    def _init():
        out_ref[...] = part

    @pl.when(hb != 0)
    def _acc():
        out_ref[...] += part


def _expert_ff(g0, g1, w1, w2, w3):
    return pl.pallas_call(
        _ff_kernel,
        grid=(E, NHB),
        in_specs=[
            pl.BlockSpec((CAP, D_MODEL), lambda e, h: (e, 0)),
            pl.BlockSpec((CAP, D_MODEL), lambda e, h: (e, 0)),
            pl.BlockSpec((1, D_MODEL, HB), lambda e, h: (e, 0, h)),
            pl.BlockSpec((1, D_MODEL, HB), lambda e, h: (e, 0, h)),
            pl.BlockSpec((1, HB, D_MODEL), lambda e, h: (e, h, 0)),
        ],
        out_specs=pl.BlockSpec((CAP, D_MODEL), lambda e, h: (e, 0)),
        out_shape=jax.ShapeDtypeStruct((E * CAP, D_MODEL), jnp.float32),
    )(g0, g1, w1, w2, w3)


def _combine_body(eo_hbm, x_hbm, c0_hbm, c1_hbm, a0_hbm, a1_hbm, bb_hbm,
                  out_hbm, xb, g0b, g1b, i0v, i1v, a0v, a1v, bbv, sem):
    c = lax.axis_index("c")
    s = lax.axis_index("s")
    wid = s * NC + c
    base = wid * (T // (NC * NS))
    for chunk in range(2):
        t0 = base + chunk * 32
        pltpu.sync_copy(x_hbm.at[pl.ds(t0, 32)], xb)
        pltpu.sync_copy(c0_hbm.at[pl.ds(t0, 32)], i0v)
        pltpu.sync_copy(c1_hbm.at[pl.ds(t0, 32)], i1v)
        pltpu.sync_copy(a0_hbm.at[pl.ds(t0, 32)], a0v)
        pltpu.sync_copy(a1_hbm.at[pl.ds(t0, 32)], a1v)
        pltpu.sync_copy(bb_hbm.at[pl.ds(t0, 32)], bbv)
        pltpu.async_copy(eo_hbm.at[i0v], g0b, sem).wait()
        pltpu.async_copy(eo_hbm.at[i1v], g1b, sem).wait()

        def row(i, carry):
            a0 = a0v[i, pl.ds(0, 16)]
            a1 = a1v[i, pl.ds(0, 16)]
            bb = bbv[i, pl.ds(0, 16)]
            for j in range(D_MODEL // 16):
                sl = pl.ds(j * 16, 16)
                xb[i, sl] = (a0 * g0b[i, sl] + a1 * g1b[i, sl]
                             + bb * xb[i, sl])
            return carry

        lax.fori_loop(0, 32, row, 0)
        pltpu.sync_copy(xb, out_hbm.at[pl.ds(t0, 32)])


def _combine(eo, x2d, comb0, comb1, a0, a1, bb):
    return pl.kernel(
        _combine_body,
        out_type=jax.ShapeDtypeStruct((T, D_MODEL), jnp.float32),
        mesh=_sc_mesh,
        scratch_types=[
            pltpu.VMEM((32, D_MODEL), jnp.float32),
            pltpu.VMEM((32, D_MODEL), jnp.float32),
            pltpu.VMEM((32, D_MODEL), jnp.float32),
            pltpu.VMEM((32,), jnp.int32),
            pltpu.VMEM((32,), jnp.int32),
            pltpu.VMEM((32, 16), jnp.float32),
            pltpu.VMEM((32, 16), jnp.float32),
            pltpu.VMEM((32, 16), jnp.float32),
            pltpu.SemaphoreType.DMA,
        ],
    )(eo, x2d, comb0, comb1, a0, a1, bb)


def kernel(x, router_w, router_b, w1, w2, w3):
    x2d = x.reshape(T, D_MODEL)
    idx, a0, a1, bb, src0, src1 = _routing(x2d, router_w, router_b)
    x_ext = jnp.concatenate([x2d, jnp.zeros((128, D_MODEL), jnp.float32)])
    g0, g1 = _dispatch(x_ext, src0.reshape(-1), src1.reshape(-1))
    eo = _expert_ff(g0, g1, w1, w2, w3)
    out = _combine(eo, x2d, idx[0], idx[1], a0, a1, bb)
    return out.reshape(1, T, D_MODEL)


# speedup vs baseline: 1.8502x; 1.0018x over previous
import jax
import jax.numpy as jnp
from jax import lax
from jax.experimental import pallas as pl
from jax.experimental.pallas import tpu as pltpu
from jax.experimental.pallas import tpu_sc as plsc

E = 8
D_MODEL = 1024
HIDDEN = 2048
T = 2048
CAP = 512
NSLOT = CAP - 1
TRASH = NSLOT

NC = 2
NS = 16
EPC = E // NC
SCR = EPC * CAP
TPT = T // NS
CHUNK = 64

_sc_mesh = plsc.VectorSubcoreMesh(core_axis_name="c", subcore_axis_name="s")


def _routing_kernel(x_ref, w_ref, b_ref, idx_ref, a0_ref, a1_ref, bb_ref,
                    s0_ref, s1_ref):
    x = x_ref[...]
    w = w_ref[...]
    scores = lax.dot_general(
        w, x, (((0,), (1,)), ((), ())),
        preferred_element_type=jnp.float32,
    ) + b_ref[...].reshape(E, 1)

    eidx = lax.broadcasted_iota(jnp.int32, (E, T), 0)
    v0 = jnp.max(scores, axis=0, keepdims=True)
    i0 = jnp.min(jnp.where(scores == v0, eidx, E), axis=0, keepdims=True)
    masked = jnp.where(eidx == i0, -jnp.inf, scores)
    v1 = jnp.max(masked, axis=0, keepdims=True)
    i1 = jnp.min(jnp.where(masked == v1, eidx, E), axis=0, keepdims=True)
    ed = jnp.exp(v1 - v0)
    denom = 1.0 + ed
    s0 = 1.0 / denom
    s1 = ed / denom

    oh0 = (eidx == i0).astype(jnp.float32)
    oh1 = (eidx == i1).astype(jnp.float32)
    i0 = i0.astype(jnp.float32)
    i1 = i1.astype(jnp.float32)
    c = jnp.concatenate([oh0, oh1], axis=0)
    k = 1
    while k < T:
        shifted = jnp.concatenate(
            [jnp.zeros((2 * E, k), jnp.float32), c[:, : T - k]], axis=1)
        c = c + shifted
        k *= 2
    pos0 = c[:E, :]
    pos1 = pos0 + c[E:, :]
    p0 = jnp.sum(oh0 * pos0, axis=0, keepdims=True)
    p1 = jnp.sum(oh1 * pos1, axis=0, keepdims=True)
    m0 = p0 < float(CAP)
    m1 = p1 < float(CAP)

    tix = lax.broadcasted_iota(jnp.int32, (1, T), 1).astype(jnp.float32)
    slot0 = i0 * float(CAP) + p0 - 1.0
    slot1 = i1 * float(CAP) + p1 - 1.0
    r0 = jnp.where(m0, slot0, tix)
    r1 = jnp.where(m1, slot1, tix)
    rd0 = jnp.where(m0, slot0, float(E * CAP))
    rd1 = jnp.where(m1, slot1, float(E * CAP))
    a0 = jnp.where(m0, s0, 0.0)
    a1 = jnp.where(m1, s1, 0.0)
    bb = jnp.where(m0, 0.0, s0) + jnp.where(m1, 0.0, s1)

    idx_ref[...] = jnp.concatenate([r0, r1], axis=0).astype(jnp.int32)
    for ref, val in ((a0_ref, a0), (a1_ref, a1), (bb_ref, bb)):
        ref[...] = jnp.broadcast_to(lax.transpose(val, (1, 0)), (T, 16))

    tw = (float(T)
          - lax.broadcasted_iota(jnp.int32, (1, T), 1).astype(jnp.float32))
    for a in range(E * CAP // 128):
        rid = (lax.broadcasted_iota(jnp.int32, (128, 1), 0).astype(jnp.float32)
               + float(a * 128))
        s0 = float(T) - jnp.sum(jnp.where(rid == rd0, tw, 0.0),
                                axis=1, keepdims=True)
        s1 = float(T) - jnp.sum(jnp.where(rid == rd1, tw, 0.0),
                                axis=1, keepdims=True)
        s0 = jnp.clip(s0, 0.0, float(T))
        s1 = jnp.clip(s1, 0.0, float(T))
        local = rid - float(a * 128)
        s0 = jnp.where(s0 == float(T), s0 + local, s0)
        s1 = jnp.where(s1 == float(T), s1 + local, s1)
        sl = pl.ds(a * 128, 128)
        s0_ref[sl, :] = s0.astype(jnp.int32)
        s1_ref[sl, :] = s1.astype(jnp.int32)


def _routing(x2d, router_w, router_b):
    return pl.pallas_call(
        _routing_kernel,
        out_shape=(
            jax.ShapeDtypeStruct((2, T), jnp.int32),
            jax.ShapeDtypeStruct((T, 16), jnp.float32),
            jax.ShapeDtypeStruct((T, 16), jnp.float32),
            jax.ShapeDtypeStruct((T, 16), jnp.float32),
            jax.ShapeDtypeStruct((E * CAP, 1), jnp.int32),
            jax.ShapeDtypeStruct((E * CAP, 1), jnp.int32),
        ),
    )(x2d, router_w, router_b)


RPT = E * CAP // (NC * NS)
DCH = 16


def _dispatch_body(xe_hbm, s0_hbm, s1_hbm, o0_hbm, o1_hbm,
                   i0v, i1v, g0a, g0b, g1a, g1b, sem0, sem1):
    c = lax.axis_index("c")
    s = lax.axis_index("s")
    base = (s * NC + c) * RPT
    nch = RPT // DCH
    g0 = (g0a, g0b)
    g1 = (g1a, g1b)
    sems = (sem0, sem1)

    pltpu.sync_copy(s0_hbm.at[pl.ds(base, RPT)], i0v)
    pltpu.sync_copy(s1_hbm.at[pl.ds(base, RPT)], i1v)

    def fire(chunk):
        p = chunk % 2
        sl = pl.ds(chunk * DCH, DCH)
        c0 = pltpu.async_copy(xe_hbm.at[i0v.at[sl]], g0[p], sems[p])
        c1 = pltpu.async_copy(xe_hbm.at[i1v.at[sl]], g1[p], sems[p])
        return c0, c1

    pend = fire(0)
    for chunk in range(nch):
        p = chunk % 2
        cur = pend
        if chunk + 1 < nch:
            pend = fire(chunk + 1)
        cur[0].wait()
        cur[1].wait()
        r0 = base + chunk * DCH
        pltpu.sync_copy(g0[p], o0_hbm.at[pl.ds(r0, DCH)])
        pltpu.sync_copy(g1[p], o1_hbm.at[pl.ds(r0, DCH)])


def _dispatch(x_ext, src0, src1):
    return pl.kernel(
        _dispatch_body,
        out_type=(
            jax.ShapeDtypeStruct((E * CAP, D_MODEL), jnp.float32),
            jax.ShapeDtypeStruct((E * CAP, D_MODEL), jnp.float32),
        ),
        mesh=_sc_mesh,
        scratch_types=[
            pltpu.VMEM((RPT,), jnp.int32),
            pltpu.VMEM((RPT,), jnp.int32),
            pltpu.VMEM((DCH, D_MODEL), jnp.float32),
            pltpu.VMEM((DCH, D_MODEL), jnp.float32),
            pltpu.VMEM((DCH, D_MODEL), jnp.float32),
            pltpu.VMEM((DCH, D_MODEL), jnp.float32),
            pltpu.SemaphoreType.DMA,
            pltpu.SemaphoreType.DMA,
        ],
    )(x_ext, src0, src1)


HB = 1024
NHB = HIDDEN // HB


def _ff_kernel(g0_ref, g1_ref, w1_ref, w2_ref, w3_ref, out_ref):
    hb = pl.program_id(1)
    g = (g0_ref[...] + g1_ref[...]).astype(jnp.bfloat16)
    w1b = w1_ref[0].astype(jnp.bfloat16)
    w2b = w2_ref[0].astype(jnp.bfloat16)
    w3b = w3_ref[0].astype(jnp.bfloat16)
    h = jnp.dot(g, w2b, preferred_element_type=jnp.float32) * jnp.dot(
        g, w1b, preferred_element_type=jnp.float32)
    h = jax.nn.gelu(h).astype(jnp.bfloat16)
    part = jnp.dot(h, w3b, preferred_element_type=jnp.float32)

    @pl.when(hb == 0)
    def _init():
        out_ref[...] = part

    @pl.when(hb != 0)
    def _acc():
        out_ref[...] += part


def _expert_ff(g0, g1, w1, w2, w3):
    return pl.pallas_call(
        _ff_kernel,
        grid=(E, NHB),
        in_specs=[
            pl.BlockSpec((CAP, D_MODEL), lambda e, h: (e, 0)),
            pl.BlockSpec((CAP, D_MODEL), lambda e, h: (e, 0)),
            pl.BlockSpec((1, D_MODEL, HB), lambda e, h: (e, 0, h)),
            pl.BlockSpec((1, D_MODEL, HB), lambda e, h: (e, 0, h)),
            pl.BlockSpec((1, HB, D_MODEL), lambda e, h: (e, h, 0)),
        ],
        out_specs=pl.BlockSpec((CAP, D_MODEL), lambda e, h: (e, 0)),
        out_shape=jax.ShapeDtypeStruct((E * CAP, D_MODEL), jnp.float32),
    )(g0, g1, w1, w2, w3)


def _combine_body(eo_hbm, x_hbm, c0_hbm, c1_hbm, a0_hbm, a1_hbm, bb_hbm,
                  out_hbm, xb, g0b, g1b, i0v, i1v, a0v, a1v, bbv, sem):
    c = lax.axis_index("c")
    s = lax.axis_index("s")
    wid = s * NC + c
    base = wid * (T // (NC * NS))
    for chunk in range(2):
        t0 = base + chunk * 32
        pltpu.sync_copy(x_hbm.at[pl.ds(t0, 32)], xb)
        pltpu.sync_copy(c0_hbm.at[pl.ds(t0, 32)], i0v)
        pltpu.sync_copy(c1_hbm.at[pl.ds(t0, 32)], i1v)
        pltpu.sync_copy(a0_hbm.at[pl.ds(t0, 32)], a0v)
        pltpu.sync_copy(a1_hbm.at[pl.ds(t0, 32)], a1v)
        pltpu.sync_copy(bb_hbm.at[pl.ds(t0, 32)], bbv)
        pltpu.async_copy(eo_hbm.at[i0v], g0b, sem).wait()
        pltpu.async_copy(eo_hbm.at[i1v], g1b, sem).wait()

        def row(i, carry):
            a0 = a0v[i, pl.ds(0, 16)]
            a1 = a1v[i, pl.ds(0, 16)]
            bb = bbv[i, pl.ds(0, 16)]
            for j in range(D_MODEL // 16):
                sl = pl.ds(j * 16, 16)
                xb[i, sl] = (a0 * g0b[i, sl] + a1 * g1b[i, sl]
                             + bb * xb[i, sl])
            return carry

        lax.fori_loop(0, 32, row, 0)
        pltpu.sync_copy(xb, out_hbm.at[pl.ds(t0, 32)])


def _combine(eo, x2d, comb0, comb1, a0, a1, bb):
    return pl.kernel(
        _combine_body,
        out_type=jax.ShapeDtypeStruct((T, D_MODEL), jnp.float32),
        mesh=_sc_mesh,
        scratch_types=[
            pltpu.VMEM((32, D_MODEL), jnp.float32),
            pltpu.VMEM((32, D_MODEL), jnp.float32),
            pltpu.VMEM((32, D_MODEL), jnp.float32),
            pltpu.VMEM((32,), jnp.int32),
            pltpu.VMEM((32,), jnp.int32),
            pltpu.VMEM((32, 16), jnp.float32),
            pltpu.VMEM((32, 16), jnp.float32),
            pltpu.VMEM((32, 16), jnp.float32),
            pltpu.SemaphoreType.DMA,
        ],
    )(eo, x2d, comb0, comb1, a0, a1, bb)


def kernel(x, router_w, router_b, w1, w2, w3):
    x2d = x.reshape(T, D_MODEL)
    idx, a0, a1, bb, src0, src1 = _routing(x2d, router_w, router_b)
    x_ext = jnp.concatenate([x2d, jnp.zeros((128, D_MODEL), jnp.float32)])
    g0, g1 = _dispatch(x_ext, src0.reshape(-1), src1.reshape(-1))
    eo = _expert_ff(g0, g1, w1, w2, w3)
    out = _combine(eo, x2d, idx[0], idx[1], a0, a1, bb)
    return out.reshape(1, T, D_MODEL)
